# Initial kernel scaffold; baseline (speedup 1.0000x reference)
import jax
import jax.numpy as jnp
from jax.experimental import pallas as pl


def kernel(x, w1, b1, w2, b2, wf1, bf1, wf2, bf2):
    raise NotImplementedError("write your pallas kernel here")


# speedup vs baseline: 3.7617x; 3.7617x over previous
import jax
import jax.numpy as jnp
from jax.experimental import pallas as pl
from jax.experimental.pallas import tpu as pltpu


_CIN1, _COUT1, _K1 = 3, 3, 5
_COUT2, _K2 = 5, 3
_H, _W = 32, 32
_OH1, _OW1 = _H - _K1 + 1, _W - _K1 + 1
_PH1, _PW1 = _OH1 // 2, _OW1 // 2
_OH2, _OW2 = _PH1 - _K2 + 1, _PW1 - _K2 + 1
_PH2, _PW2 = _OH2 // 2, _OW2 // 2
_NFEAT = _COUT2 * _PH2 * _PW2
_NHID, _NOUT = 100, 10

_RB1 = 4
_NB1 = _OH1 // _RB1
_XR1 = _RB1 + _K1 - 1
_M1 = _COUT1 * _RB1 * _OW1
_KK1 = _CIN1 * _XR1 * _W
_M2 = _COUT2 * _OH2 * _OW2
_KK2 = _CIN1 * _PH1 * _PW1


def _band_matrices(w1, w2):
    w1r = w1.reshape(_COUT1, _CIN1, _K1, _K1)
    w2r = w2.reshape(_COUT2, _CIN1, _K2, _K2)

    jrow1 = (2 * jnp.arange(_RB1 // 2)[None, :, None]
             + jnp.arange(2)[:, None, None])
    a1 = (jnp.arange(_XR1)[None, None, :, None] - jrow1[..., None]
          == jnp.arange(_K1)[None, None, None, :])
    b1m = (jnp.arange(_W)[None, :, None] - jnp.arange(_OW1)[:, None, None]
           == jnp.arange(_K1)[None, None, :])
    band1 = jnp.einsum("oikl,pjhk,qwl->opjqihw",
                       w1r, a1.astype(w1.dtype), b1m.astype(w1.dtype))
    band1 = band1.reshape(_M1, _KK1)

    jrow2 = (2 * jnp.arange(_OH2 // 2)[None, :, None]
             + jnp.arange(2)[:, None, None])
    a2 = (jnp.arange(_PH1)[None, None, :, None] - jrow2[..., None]
          == jnp.arange(_K2)[None, None, None, :])
    b2m = (jnp.arange(_PW1)[None, :, None] - jnp.arange(_OW2)[:, None, None]
           == jnp.arange(_K2)[None, None, :])
    band2 = jnp.einsum("oikl,pjhk,qwl->opjqihw",
                       w2r, a2.astype(w2.dtype), b2m.astype(w2.dtype))
    band2 = band2.reshape(_M2, _KK2)
    return band1, band2


def _net_kernel(x_ref, wb1_ref, b1_ref, wb2_ref, b2_ref,
                wf1_ref, bf1_ref, wf2_ref, bf2_ref,
                o_ref, p1_ref):
    tn = x_ref.shape[-1]

    b1c = b1_ref[...].reshape(_COUT1, 1, 1, 1)
    for blk in range(_NB1):
        xs = x_ref[:, _RB1 * blk:_RB1 * blk + _XR1, :, :].reshape(_KK1, tn)
        z = jnp.dot(wb1_ref[...], xs, preferred_element_type=jnp.float32)
        z = z.reshape(_COUT1, 2, (_RB1 // 2) * _OW1, tn)
        zh = jnp.maximum(z[:, 0], z[:, 1])
        zh = zh.reshape(_COUT1, _RB1 // 2, _PW1, 2, tn)
        zp = jnp.maximum(zh[:, :, :, 0], zh[:, :, :, 1])
        p1_ref[:, 2 * blk:2 * blk + 2, :, :] = jnp.maximum(zp + b1c, 0.0)

    f1 = p1_ref[...].reshape(_KK2, tn)
    z2 = jnp.dot(wb2_ref[...], f1, preferred_element_type=jnp.float32)
    z2 = z2.reshape(_COUT2, 2, _PH2 * _OW2, tn)
    zh2 = jnp.maximum(z2[:, 0], z2[:, 1])
    zh2 = zh2.reshape(_COUT2, _PH2, _PW2, 2, tn)
    zp2 = jnp.maximum(zh2[:, :, :, 0], zh2[:, :, :, 1])
    b2c = b2_ref[...].reshape(_COUT2, 1, 1, 1)
    feats = jnp.maximum(zp2 + b2c, 0.0).reshape(_NFEAT, tn)

    h = jnp.dot(wf1_ref[...], feats, preferred_element_type=jnp.float32)
    h = jnp.maximum(h + bf1_ref[...], 0.0)
    o = jnp.dot(wf2_ref[...], h, preferred_element_type=jnp.float32)
    o_ref[...] = o + bf2_ref[...]


def kernel(x, w1, b1, w2, b2, wf1, bf1, wf2, bf2):
    band1, band2 = _band_matrices(w1, w2)

    n = x.shape[0]
    tile_n = n if n <= 256 else 256
    n_pad = ((n + tile_n - 1) // tile_n) * tile_n

    x_t = jnp.transpose(x, (1, 2, 3, 0)).astype(jnp.float32)
    if n_pad != n:
        x_t = jnp.pad(x_t, ((0, 0), (0, 0), (0, 0), (0, n_pad - n)))

    out = pl.pallas_call(
        _net_kernel,
        out_shape=jax.ShapeDtypeStruct((_NOUT, n_pad), jnp.float32),
        grid=(n_pad // tile_n,),
        in_specs=[
            pl.BlockSpec((_CIN1, _H, _W, tile_n), lambda i: (0, 0, 0, i)),
            pl.BlockSpec((_M1, _KK1), lambda i: (0, 0)),
            pl.BlockSpec((_COUT1, 1), lambda i: (0, 0)),
            pl.BlockSpec((_M2, _KK2), lambda i: (0, 0)),
            pl.BlockSpec((_COUT2, 1), lambda i: (0, 0)),
            pl.BlockSpec((_NHID, _NFEAT), lambda i: (0, 0)),
            pl.BlockSpec((_NHID, 1), lambda i: (0, 0)),
            pl.BlockSpec((_NOUT, _NHID), lambda i: (0, 0)),
            pl.BlockSpec((_NOUT, 1), lambda i: (0, 0)),
        ],
        out_specs=pl.BlockSpec((_NOUT, tile_n), lambda i: (0, i)),
        scratch_shapes=[
            pltpu.VMEM((_CIN1, _PH1, _PW1, tile_n), jnp.float32),
        ],
        compiler_params=pltpu.CompilerParams(
            dimension_semantics=("parallel",),
            vmem_limit_bytes=48 * 1024 * 1024,
        ),
    )(x_t, band1, b1.reshape(_COUT1, 1), band2, b2.reshape(_COUT2, 1),
      wf1, bf1, wf2, bf2)

    return out[:, :n].T
